# baseline (device time: 115779 ns/iter reference)
import jax
import jax.numpy as jnp
from jax import lax
from jax.experimental import pallas as pl
from jax.experimental.pallas import tpu as pltpu

N_DEV = 4
SQ = 2048
SKV = 2048
D_MODEL = 1024
H_PER = 8
DH = 128
BQ = 512
NB = SQ // BQ
QH = BQ // 2
SCALE = 0.08838834764831843
BLK = 64


def _prep_body(idx_ref, x_ref, wq_ref, kslab_ref, vslab_ref,
               q_ref, k8_ref, v8_ref):
    x = x_ref[...].astype(jnp.bfloat16)
    wq = wq_ref[...].astype(jnp.bfloat16)
    q_ref[...] = (
        jnp.dot(x, wq, preferred_element_type=jnp.float32) * SCALE
    ).astype(jnp.bfloat16)
    for h in range(H_PER):
        k8_ref[h] = kslab_ref[:, h, :].astype(jnp.bfloat16)
        v8_ref[h] = vslab_ref[:, h, :].astype(jnp.bfloat16)


def _mega_body(q_ref, k_ref, v_ref, wo_ref, out_ref,
               ctx_ref, p_ref, sbuf_ref, rbuf1_ref, rbuf2_ref,
               s1_send, s1_recv, s2_send, s2_recv):
    my = lax.axis_index("i")
    partner_a = my ^ 1
    partner_b = 3 - my

    barrier = pltpu.get_barrier_semaphore()
    for nbr in (partner_a, partner_b):
        pl.semaphore_signal(
            barrier, inc=1, device_id=(nbr,),
            device_id_type=pl.DeviceIdType.MESH,
        )
    pl.semaphore_wait(barrier, 2)

    wo = wo_ref[...].astype(jnp.bfloat16)

    stage1 = {}
    stage2 = {}

    def start_stage1(qi):
        r0 = qi * BQ
        ra = pltpu.make_async_remote_copy(
            src_ref=p_ref.at[pl.ds(r0, QH), :],
            dst_ref=rbuf1_ref.at[qi, 0],
            send_sem=s1_send.at[qi, 0],
            recv_sem=s1_recv.at[qi, 0],
            device_id=(partner_a,),
            device_id_type=pl.DeviceIdType.MESH,
        )
        rb = pltpu.make_async_remote_copy(
            src_ref=p_ref.at[pl.ds(r0 + QH, QH), :],
            dst_ref=rbuf1_ref.at[qi, 1],
            send_sem=s1_send.at[qi, 1],
            recv_sem=s1_recv.at[qi, 1],
            device_id=(partner_b,),
            device_id_type=pl.DeviceIdType.MESH,
        )
        ra.start()
        rb.start()
        stage1[qi] = (ra, rb)

    def finish1_start2(qi):
        ra, rb = stage1[qi]
        ra.wait()
        rb.wait()
        r0 = qi * BQ
        sbuf_ref[qi, 0] = p_ref[pl.ds(r0, QH), :] + rbuf1_ref[qi, 0]
        sbuf_ref[qi, 1] = p_ref[pl.ds(r0 + QH, QH), :] + rbuf1_ref[qi, 1]
        ra2 = pltpu.make_async_remote_copy(
            src_ref=sbuf_ref.at[qi, 0],
            dst_ref=rbuf2_ref.at[qi, 0],
            send_sem=s2_send.at[qi, 0],
            recv_sem=s2_recv.at[qi, 0],
            device_id=(partner_b,),
            device_id_type=pl.DeviceIdType.MESH,
        )
        rb2 = pltpu.make_async_remote_copy(
            src_ref=sbuf_ref.at[qi, 1],
            dst_ref=rbuf2_ref.at[qi, 1],
            send_sem=s2_send.at[qi, 1],
            recv_sem=s2_recv.at[qi, 1],
            device_id=(partner_a,),
            device_id_type=pl.DeviceIdType.MESH,
        )
        ra2.start()
        rb2.start()
        stage2[qi] = (ra2, rb2)

    for qi in range(NB):
        r0 = qi * BQ
        for h in range(H_PER):
            q = q_ref[pl.ds(r0, BQ), pl.ds(h * DH, DH)]
            l = jnp.zeros((BQ, 1), jnp.float32)
            acc = jnp.zeros((BQ, DH), jnp.float32)
            for j in range(qi):
                kc = k_ref[h, pl.ds(j * BQ, BQ), :]
                sc = lax.dot_general(
                    q, kc, (((1,), (1,)), ((), ())),
                    preferred_element_type=jnp.float32,
                )
                pc = jnp.exp(sc)
                l = l + jnp.sum(pc, axis=1, keepdims=True)
                acc = acc + jnp.dot(
                    pc.astype(jnp.bfloat16),
                    v_ref[h, pl.ds(j * BQ, BQ), :],
                    preferred_element_type=jnp.float32,
                )
            kd = k_ref[h, pl.ds(r0, BQ), :]
            s_d = lax.dot_general(
                q, kd, (((1,), (1,)), ((), ())),
                preferred_element_type=jnp.float32,
            )
            rowb = lax.broadcasted_iota(jnp.int32, (BQ, BQ), 0) // BLK
            colb = lax.broadcasted_iota(jnp.int32, (BQ, BQ), 1) // BLK
            p_d = jnp.where(colb <= rowb, jnp.exp(s_d), 0.0)
            l = l + jnp.sum(p_d, axis=1, keepdims=True)
            acc = acc + jnp.dot(
                p_d.astype(jnp.bfloat16),
                v_ref[h, pl.ds(r0, BQ), :],
                preferred_element_type=jnp.float32,
            )
            ctx_ref[:, pl.ds(h * DH, DH)] = (acc / l).astype(jnp.bfloat16)

        p_ref[pl.ds(r0, BQ), :] = jnp.dot(
            ctx_ref[...], wo, preferred_element_type=jnp.float32
        ).astype(jnp.bfloat16)
        start_stage1(qi)
        if qi > 0:
            finish1_start2(qi - 1)

    finish1_start2(NB - 1)

    for qi in range(NB):
        ra2, rb2 = stage2[qi]
        ra2.wait()
        rb2.wait()
        r0 = qi * BQ
        out_ref[pl.ds(r0, QH), :] = (sbuf_ref[qi, 0] + rbuf2_ref[qi, 0])
        out_ref[pl.ds(r0 + QH, QH), :] = (sbuf_ref[qi, 1] + rbuf2_ref[qi, 1])


def kernel(x, Wq, K_ext, V_ext, Wo):
    my = lax.axis_index("i")
    x2 = x.reshape(SQ, D_MODEL)
    my_idx = my.astype(jnp.int32).reshape((1,))

    Q, K, V = pl.pallas_call(
        _prep_body,
        grid_spec=pltpu.PrefetchScalarGridSpec(
            num_scalar_prefetch=1,
            grid=(1,),
            in_specs=[
                pl.BlockSpec((SQ, D_MODEL), lambda i, s: (0, 0)),
                pl.BlockSpec((D_MODEL, D_MODEL), lambda i, s: (0, 0)),
                pl.BlockSpec((SKV, H_PER, DH), lambda i, s: (0, s[0], 0)),
                pl.BlockSpec((SKV, H_PER, DH), lambda i, s: (0, s[0], 0)),
            ],
            out_specs=[
                pl.BlockSpec((SQ, D_MODEL), lambda i, s: (0, 0)),
                pl.BlockSpec((H_PER, SKV, DH), lambda i, s: (0, 0, 0)),
                pl.BlockSpec((H_PER, SKV, DH), lambda i, s: (0, 0, 0)),
            ],
        ),
        out_shape=[
            jax.ShapeDtypeStruct((SQ, D_MODEL), jnp.bfloat16),
            jax.ShapeDtypeStruct((H_PER, SKV, DH), jnp.bfloat16),
            jax.ShapeDtypeStruct((H_PER, SKV, DH), jnp.bfloat16),
        ],
    )(my_idx, x2, Wq, K_ext.reshape(SKV, 32, DH), V_ext.reshape(SKV, 32, DH))

    out = pl.pallas_call(
        _mega_body,
        out_shape=jax.ShapeDtypeStruct((SQ, D_MODEL), jnp.bfloat16),
        in_specs=[
            pl.BlockSpec(memory_space=pltpu.VMEM),
            pl.BlockSpec(memory_space=pltpu.VMEM),
            pl.BlockSpec(memory_space=pltpu.VMEM),
            pl.BlockSpec(memory_space=pltpu.VMEM),
        ],
        out_specs=pl.BlockSpec(memory_space=pltpu.VMEM),
        scratch_shapes=[
            pltpu.VMEM((BQ, H_PER * DH), jnp.bfloat16),
            pltpu.VMEM((SQ, D_MODEL), jnp.bfloat16),
            pltpu.VMEM((NB, 2, QH, D_MODEL), jnp.bfloat16),
            pltpu.VMEM((NB, 2, QH, D_MODEL), jnp.bfloat16),
            pltpu.VMEM((NB, 2, QH, D_MODEL), jnp.bfloat16),
            pltpu.SemaphoreType.DMA((NB, 2)),
            pltpu.SemaphoreType.DMA((NB, 2)),
            pltpu.SemaphoreType.DMA((NB, 2)),
            pltpu.SemaphoreType.DMA((NB, 2)),
        ],
        compiler_params=pltpu.CompilerParams(collective_id=0),
    )(Q, K, V, Wo)

    return out.reshape(1, SQ, D_MODEL)


# device time: 84043 ns/iter; 1.3776x vs baseline; 1.3776x over previous
import jax
import jax.numpy as jnp
from jax import lax
from jax.experimental import pallas as pl
from jax.experimental.pallas import tpu as pltpu

N_DEV = 4
SQ = 2048
SKV = 2048
D_MODEL = 1024
H_PER = 8
DH = 128
BQ = 512
NB = SQ // BQ
QH = BQ // 2
SCALE = 0.08838834764831843
BLK = 64


def _prep_body(idx_ref, x_ref, wq_ref, kslab_ref, vslab_ref,
               q_ref, k8_ref, v8_ref):
    x = x_ref[...].astype(jnp.bfloat16)
    wq = wq_ref[...].astype(jnp.bfloat16)
    q_ref[...] = (
        jnp.dot(x, wq, preferred_element_type=jnp.float32) * SCALE
    ).astype(jnp.bfloat16)
    k8_ref[...] = jnp.transpose(kslab_ref[...].astype(jnp.bfloat16), (1, 0, 2))
    v8_ref[...] = jnp.transpose(vslab_ref[...].astype(jnp.bfloat16), (1, 0, 2))


def _mega_body(q_ref, k_ref, v_ref, wo_ref, out_ref,
               ctx_ref, p_ref, sbuf_ref, rbuf1_ref, rbuf2_ref,
               s1_send, s1_recv, s2_send, s2_recv):
    my = lax.axis_index("i")
    partner_a = my ^ 1
    partner_b = 3 - my

    barrier = pltpu.get_barrier_semaphore()
    for nbr in (partner_a, partner_b):
        pl.semaphore_signal(
            barrier, inc=1, device_id=(nbr,),
            device_id_type=pl.DeviceIdType.MESH,
        )
    pl.semaphore_wait(barrier, 2)

    wo = wo_ref[...].astype(jnp.bfloat16)

    stage1 = {}
    stage2 = {}

    def start_stage1(qi):
        r0 = qi * BQ
        ra = pltpu.make_async_remote_copy(
            src_ref=p_ref.at[pl.ds(r0, QH), :],
            dst_ref=rbuf1_ref.at[qi, 0],
            send_sem=s1_send.at[qi, 0],
            recv_sem=s1_recv.at[qi, 0],
            device_id=(partner_a,),
            device_id_type=pl.DeviceIdType.MESH,
        )
        rb = pltpu.make_async_remote_copy(
            src_ref=p_ref.at[pl.ds(r0 + QH, QH), :],
            dst_ref=rbuf1_ref.at[qi, 1],
            send_sem=s1_send.at[qi, 1],
            recv_sem=s1_recv.at[qi, 1],
            device_id=(partner_b,),
            device_id_type=pl.DeviceIdType.MESH,
        )
        ra.start()
        rb.start()
        stage1[qi] = (ra, rb)

    def finish1_start2(qi):
        ra, rb = stage1[qi]
        ra.wait()
        rb.wait()
        r0 = qi * BQ
        sbuf_ref[qi, 0] = p_ref[pl.ds(r0, QH), :] + rbuf1_ref[qi, 0]
        sbuf_ref[qi, 1] = p_ref[pl.ds(r0 + QH, QH), :] + rbuf1_ref[qi, 1]
        ra2 = pltpu.make_async_remote_copy(
            src_ref=sbuf_ref.at[qi, 0],
            dst_ref=rbuf2_ref.at[qi, 0],
            send_sem=s2_send.at[qi, 0],
            recv_sem=s2_recv.at[qi, 0],
            device_id=(partner_b,),
            device_id_type=pl.DeviceIdType.MESH,
        )
        rb2 = pltpu.make_async_remote_copy(
            src_ref=sbuf_ref.at[qi, 1],
            dst_ref=rbuf2_ref.at[qi, 1],
            send_sem=s2_send.at[qi, 1],
            recv_sem=s2_recv.at[qi, 1],
            device_id=(partner_a,),
            device_id_type=pl.DeviceIdType.MESH,
        )
        ra2.start()
        rb2.start()
        stage2[qi] = (ra2, rb2)

    for qi in range(NB):
        r0 = qi * BQ
        for h in range(H_PER):
            q = q_ref[pl.ds(r0, BQ), pl.ds(h * DH, DH)]
            l = jnp.zeros((BQ, 1), jnp.float32)
            acc = jnp.zeros((BQ, DH), jnp.float32)
            for j in range(qi):
                kc = k_ref[h, pl.ds(j * BQ, BQ), :]
                sc = lax.dot_general(
                    q, kc, (((1,), (1,)), ((), ())),
                    preferred_element_type=jnp.float32,
                )
                pc = jnp.exp(sc)
                l = l + jnp.sum(pc, axis=1, keepdims=True)
                acc = acc + jnp.dot(
                    pc.astype(jnp.bfloat16),
                    v_ref[h, pl.ds(j * BQ, BQ), :],
                    preferred_element_type=jnp.float32,
                )
            kd = k_ref[h, pl.ds(r0, BQ), :]
            s_d = lax.dot_general(
                q, kd, (((1,), (1,)), ((), ())),
                preferred_element_type=jnp.float32,
            )
            rowb = lax.broadcasted_iota(jnp.int32, (BQ, BQ), 0) // BLK
            colb = lax.broadcasted_iota(jnp.int32, (BQ, BQ), 1) // BLK
            p_d = jnp.where(colb <= rowb, jnp.exp(s_d), 0.0)
            l = l + jnp.sum(p_d, axis=1, keepdims=True)
            acc = acc + jnp.dot(
                p_d.astype(jnp.bfloat16),
                v_ref[h, pl.ds(r0, BQ), :],
                preferred_element_type=jnp.float32,
            )
            ctx_ref[:, pl.ds(h * DH, DH)] = (acc / l).astype(jnp.bfloat16)

        p_ref[pl.ds(r0, BQ), :] = jnp.dot(
            ctx_ref[...], wo, preferred_element_type=jnp.float32
        ).astype(jnp.bfloat16)
        start_stage1(qi)
        if qi > 0:
            finish1_start2(qi - 1)

    finish1_start2(NB - 1)

    for qi in range(NB):
        ra2, rb2 = stage2[qi]
        ra2.wait()
        rb2.wait()
        r0 = qi * BQ
        out_ref[pl.ds(r0, QH), :] = (sbuf_ref[qi, 0] + rbuf2_ref[qi, 0])
        out_ref[pl.ds(r0 + QH, QH), :] = (sbuf_ref[qi, 1] + rbuf2_ref[qi, 1])


def kernel(x, Wq, K_ext, V_ext, Wo):
    my = lax.axis_index("i")
    x2 = x.reshape(SQ, D_MODEL)
    my_idx = my.astype(jnp.int32).reshape((1,))

    Q, K, V = pl.pallas_call(
        _prep_body,
        grid_spec=pltpu.PrefetchScalarGridSpec(
            num_scalar_prefetch=1,
            grid=(1,),
            in_specs=[
                pl.BlockSpec((SQ, D_MODEL), lambda i, s: (0, 0)),
                pl.BlockSpec((D_MODEL, D_MODEL), lambda i, s: (0, 0)),
                pl.BlockSpec((SKV, H_PER, DH), lambda i, s: (0, s[0], 0)),
                pl.BlockSpec((SKV, H_PER, DH), lambda i, s: (0, s[0], 0)),
            ],
            out_specs=[
                pl.BlockSpec((SQ, D_MODEL), lambda i, s: (0, 0)),
                pl.BlockSpec((H_PER, SKV, DH), lambda i, s: (0, 0, 0)),
                pl.BlockSpec((H_PER, SKV, DH), lambda i, s: (0, 0, 0)),
            ],
        ),
        out_shape=[
            jax.ShapeDtypeStruct((SQ, D_MODEL), jnp.bfloat16),
            jax.ShapeDtypeStruct((H_PER, SKV, DH), jnp.bfloat16),
            jax.ShapeDtypeStruct((H_PER, SKV, DH), jnp.bfloat16),
        ],
    )(my_idx, x2, Wq, K_ext.reshape(SKV, 32, DH), V_ext.reshape(SKV, 32, DH))

    out = pl.pallas_call(
        _mega_body,
        out_shape=jax.ShapeDtypeStruct((SQ, D_MODEL), jnp.bfloat16),
        in_specs=[
            pl.BlockSpec(memory_space=pltpu.VMEM),
            pl.BlockSpec(memory_space=pltpu.VMEM),
            pl.BlockSpec(memory_space=pltpu.VMEM),
            pl.BlockSpec(memory_space=pltpu.VMEM),
        ],
        out_specs=pl.BlockSpec(memory_space=pltpu.VMEM),
        scratch_shapes=[
            pltpu.VMEM((BQ, H_PER * DH), jnp.bfloat16),
            pltpu.VMEM((SQ, D_MODEL), jnp.bfloat16),
            pltpu.VMEM((NB, 2, QH, D_MODEL), jnp.bfloat16),
            pltpu.VMEM((NB, 2, QH, D_MODEL), jnp.bfloat16),
            pltpu.VMEM((NB, 2, QH, D_MODEL), jnp.bfloat16),
            pltpu.SemaphoreType.DMA((NB, 2)),
            pltpu.SemaphoreType.DMA((NB, 2)),
            pltpu.SemaphoreType.DMA((NB, 2)),
            pltpu.SemaphoreType.DMA((NB, 2)),
        ],
        compiler_params=pltpu.CompilerParams(collective_id=0),
    )(Q, K, V, Wo)

    return out.reshape(1, SQ, D_MODEL)


# device time: 82961 ns/iter; 1.3956x vs baseline; 1.0130x over previous
import jax
import jax.numpy as jnp
from jax import lax
from jax.experimental import pallas as pl
from jax.experimental.pallas import tpu as pltpu

N_DEV = 4
SQ = 2048
SKV = 2048
D_MODEL = 1024
H_PER = 8
DH = 128
BQ = 512
NB = SQ // BQ
QH = BQ // 2
SCALE = 0.08838834764831843
BLK = 64


def _prep_body(idx_ref, x_ref, wq_ref, kslab_ref, vslab_ref,
               q_ref, k8_ref, v8_ref):
    x = x_ref[...].astype(jnp.bfloat16)
    wq = wq_ref[...].astype(jnp.bfloat16)
    q_ref[...] = (
        jnp.dot(x, wq, preferred_element_type=jnp.float32) * SCALE
    ).astype(jnp.bfloat16)
    k8_ref[...] = jnp.transpose(kslab_ref[...].astype(jnp.bfloat16), (1, 0, 2))
    v8_ref[...] = jnp.transpose(vslab_ref[...].astype(jnp.bfloat16), (1, 0, 2))


def _mega_body(q_ref, k_ref, v_ref, wo_ref, out_ref,
               ctx_ref, p_ref, sbuf_ref, rbuf1_ref, rbuf2_ref,
               s1_send, s1_recv, s2_send, s2_recv):
    my = lax.axis_index("i")
    partner_a = my ^ 1
    partner_b = 3 - my

    barrier = pltpu.get_barrier_semaphore()
    for nbr in (partner_a, partner_b):
        pl.semaphore_signal(
            barrier, inc=1, device_id=(nbr,),
            device_id_type=pl.DeviceIdType.MESH,
        )

    wo = wo_ref[...].astype(jnp.bfloat16)

    stage1 = {}
    stage2 = {}

    def start_stage1(qi):
        r0 = qi * BQ
        ra = pltpu.make_async_remote_copy(
            src_ref=p_ref.at[pl.ds(r0, QH), :],
            dst_ref=rbuf1_ref.at[qi, 0],
            send_sem=s1_send.at[qi, 0],
            recv_sem=s1_recv.at[qi, 0],
            device_id=(partner_a,),
            device_id_type=pl.DeviceIdType.MESH,
        )
        rb = pltpu.make_async_remote_copy(
            src_ref=p_ref.at[pl.ds(r0 + QH, QH), :],
            dst_ref=rbuf1_ref.at[qi, 1],
            send_sem=s1_send.at[qi, 1],
            recv_sem=s1_recv.at[qi, 1],
            device_id=(partner_b,),
            device_id_type=pl.DeviceIdType.MESH,
        )
        ra.start()
        rb.start()
        stage1[qi] = (ra, rb)

    def finish1_start2(qi):
        ra, rb = stage1[qi]
        ra.wait()
        rb.wait()
        r0 = qi * BQ
        sbuf_ref[qi, 0] = p_ref[pl.ds(r0, QH), :] + rbuf1_ref[qi, 0]
        sbuf_ref[qi, 1] = p_ref[pl.ds(r0 + QH, QH), :] + rbuf1_ref[qi, 1]
        ra2 = pltpu.make_async_remote_copy(
            src_ref=sbuf_ref.at[qi, 0],
            dst_ref=rbuf2_ref.at[qi, 0],
            send_sem=s2_send.at[qi, 0],
            recv_sem=s2_recv.at[qi, 0],
            device_id=(partner_b,),
            device_id_type=pl.DeviceIdType.MESH,
        )
        rb2 = pltpu.make_async_remote_copy(
            src_ref=sbuf_ref.at[qi, 1],
            dst_ref=rbuf2_ref.at[qi, 1],
            send_sem=s2_send.at[qi, 1],
            recv_sem=s2_recv.at[qi, 1],
            device_id=(partner_a,),
            device_id_type=pl.DeviceIdType.MESH,
        )
        ra2.start()
        rb2.start()
        stage2[qi] = (ra2, rb2)

    for qi in range(NB):
        r0 = qi * BQ
        for h in range(H_PER):
            q = q_ref[pl.ds(r0, BQ), pl.ds(h * DH, DH)]
            l = jnp.zeros((BQ, 1), jnp.float32)
            acc = jnp.zeros((BQ, DH), jnp.float32)
            for j in range(qi):
                kc = k_ref[h, pl.ds(j * BQ, BQ), :]
                sc = lax.dot_general(
                    q, kc, (((1,), (1,)), ((), ())),
                    preferred_element_type=jnp.float32,
                )
                pc = jnp.exp(sc)
                l = l + jnp.sum(pc, axis=1, keepdims=True)
                acc = acc + jnp.dot(
                    pc.astype(jnp.bfloat16),
                    v_ref[h, pl.ds(j * BQ, BQ), :],
                    preferred_element_type=jnp.float32,
                )
            kd = k_ref[h, pl.ds(r0, BQ), :]
            s_d = lax.dot_general(
                q, kd, (((1,), (1,)), ((), ())),
                preferred_element_type=jnp.float32,
            )
            rowb = lax.broadcasted_iota(jnp.int32, (BQ, BQ), 0) // BLK
            colb = lax.broadcasted_iota(jnp.int32, (BQ, BQ), 1) // BLK
            p_d = jnp.where(colb <= rowb, jnp.exp(s_d), 0.0)
            l = l + jnp.sum(p_d, axis=1, keepdims=True)
            acc = acc + jnp.dot(
                p_d.astype(jnp.bfloat16),
                v_ref[h, pl.ds(r0, BQ), :],
                preferred_element_type=jnp.float32,
            )
            ctx_ref[:, pl.ds(h * DH, DH)] = (acc / l).astype(jnp.bfloat16)

        p_ref[pl.ds(r0, BQ), :] = jnp.dot(
            ctx_ref[...], wo, preferred_element_type=jnp.float32
        ).astype(jnp.bfloat16)
        if qi == 0:
            pl.semaphore_wait(barrier, 2)
        start_stage1(qi)
        if qi > 0:
            finish1_start2(qi - 1)

    finish1_start2(NB - 1)

    for qi in range(NB):
        ra2, rb2 = stage2[qi]
        ra2.wait()
        rb2.wait()
        r0 = qi * BQ
        out_ref[pl.ds(r0, QH), :] = (sbuf_ref[qi, 0] + rbuf2_ref[qi, 0])
        out_ref[pl.ds(r0 + QH, QH), :] = (sbuf_ref[qi, 1] + rbuf2_ref[qi, 1])


def kernel(x, Wq, K_ext, V_ext, Wo):
    my = lax.axis_index("i")
    x2 = x.reshape(SQ, D_MODEL)
    my_idx = my.astype(jnp.int32).reshape((1,))

    Q, K, V = pl.pallas_call(
        _prep_body,
        grid_spec=pltpu.PrefetchScalarGridSpec(
            num_scalar_prefetch=1,
            grid=(1,),
            in_specs=[
                pl.BlockSpec((SQ, D_MODEL), lambda i, s: (0, 0)),
                pl.BlockSpec((D_MODEL, D_MODEL), lambda i, s: (0, 0)),
                pl.BlockSpec((SKV, H_PER, DH), lambda i, s: (0, s[0], 0)),
                pl.BlockSpec((SKV, H_PER, DH), lambda i, s: (0, s[0], 0)),
            ],
            out_specs=[
                pl.BlockSpec((SQ, D_MODEL), lambda i, s: (0, 0)),
                pl.BlockSpec((H_PER, SKV, DH), lambda i, s: (0, 0, 0)),
                pl.BlockSpec((H_PER, SKV, DH), lambda i, s: (0, 0, 0)),
            ],
        ),
        out_shape=[
            jax.ShapeDtypeStruct((SQ, D_MODEL), jnp.bfloat16),
            jax.ShapeDtypeStruct((H_PER, SKV, DH), jnp.bfloat16),
            jax.ShapeDtypeStruct((H_PER, SKV, DH), jnp.bfloat16),
        ],
    )(my_idx, x2, Wq, K_ext.reshape(SKV, 32, DH), V_ext.reshape(SKV, 32, DH))

    out = pl.pallas_call(
        _mega_body,
        out_shape=jax.ShapeDtypeStruct((SQ, D_MODEL), jnp.bfloat16),
        in_specs=[
            pl.BlockSpec(memory_space=pltpu.VMEM),
            pl.BlockSpec(memory_space=pltpu.VMEM),
            pl.BlockSpec(memory_space=pltpu.VMEM),
            pl.BlockSpec(memory_space=pltpu.VMEM),
        ],
        out_specs=pl.BlockSpec(memory_space=pltpu.VMEM),
        scratch_shapes=[
            pltpu.VMEM((BQ, H_PER * DH), jnp.bfloat16),
            pltpu.VMEM((SQ, D_MODEL), jnp.bfloat16),
            pltpu.VMEM((NB, 2, QH, D_MODEL), jnp.bfloat16),
            pltpu.VMEM((NB, 2, QH, D_MODEL), jnp.bfloat16),
            pltpu.VMEM((NB, 2, QH, D_MODEL), jnp.bfloat16),
            pltpu.SemaphoreType.DMA((NB, 2)),
            pltpu.SemaphoreType.DMA((NB, 2)),
            pltpu.SemaphoreType.DMA((NB, 2)),
            pltpu.SemaphoreType.DMA((NB, 2)),
        ],
        compiler_params=pltpu.CompilerParams(collective_id=0),
    )(Q, K, V, Wo)

    return out.reshape(1, SQ, D_MODEL)


# device time: 79202 ns/iter; 1.4618x vs baseline; 1.0475x over previous
import jax
import jax.numpy as jnp
from jax import lax
from jax.experimental import pallas as pl
from jax.experimental.pallas import tpu as pltpu

N_DEV = 4
SQ = 2048
SKV = 2048
D_MODEL = 1024
H_PER = 8
DH = 128
BQ = 512
NB = SQ // BQ
QH = BQ // 2
SCALE = 0.08838834764831843
BLK = 64


def _prep_body(idx_ref, x_ref, wq_ref, kslab_ref, vslab_ref,
               q_ref, k8_ref, v8_ref):
    x = x_ref[...].astype(jnp.bfloat16)
    wq = wq_ref[...].astype(jnp.bfloat16)
    q_ref[...] = (
        jnp.dot(x, wq, preferred_element_type=jnp.float32) * SCALE
    ).astype(jnp.bfloat16)
    k8_ref[...] = jnp.transpose(kslab_ref[...].astype(jnp.bfloat16), (1, 0, 2))
    v8_ref[...] = jnp.transpose(vslab_ref[...].astype(jnp.bfloat16), (1, 0, 2))


def _mega_body(q_ref, k_ref, v_ref, wo_ref, out_ref,
               ctx_ref, p_ref, sbuf_ref, rbuf1_ref, rbuf2_ref,
               s1_send, s1_recv, s2_send, s2_recv):
    my = lax.axis_index("i")
    partner_a = my ^ 1
    partner_b = 3 - my

    barrier = pltpu.get_barrier_semaphore()
    for nbr in (partner_a, partner_b):
        pl.semaphore_signal(
            barrier, inc=1, device_id=(nbr,),
            device_id_type=pl.DeviceIdType.MESH,
        )

    wo = wo_ref[...].astype(jnp.bfloat16)

    stage1 = {}
    stage2 = {}

    def start_stage1(qi):
        r0 = qi * BQ
        ra = pltpu.make_async_remote_copy(
            src_ref=p_ref.at[pl.ds(r0, QH), :],
            dst_ref=rbuf1_ref.at[qi, 0],
            send_sem=s1_send.at[qi, 0],
            recv_sem=s1_recv.at[qi, 0],
            device_id=(partner_a,),
            device_id_type=pl.DeviceIdType.MESH,
        )
        rb = pltpu.make_async_remote_copy(
            src_ref=p_ref.at[pl.ds(r0 + QH, QH), :],
            dst_ref=rbuf1_ref.at[qi, 1],
            send_sem=s1_send.at[qi, 1],
            recv_sem=s1_recv.at[qi, 1],
            device_id=(partner_b,),
            device_id_type=pl.DeviceIdType.MESH,
        )
        ra.start()
        rb.start()
        stage1[qi] = (ra, rb)

    def finish1_start2(qi):
        ra, rb = stage1[qi]
        ra.wait()
        rb.wait()
        r0 = qi * BQ
        sbuf_ref[qi, 0] = p_ref[pl.ds(r0, QH), :] + rbuf1_ref[qi, 0]
        sbuf_ref[qi, 1] = p_ref[pl.ds(r0 + QH, QH), :] + rbuf1_ref[qi, 1]
        ra2 = pltpu.make_async_remote_copy(
            src_ref=sbuf_ref.at[qi, 0],
            dst_ref=rbuf2_ref.at[qi, 0],
            send_sem=s2_send.at[qi, 0],
            recv_sem=s2_recv.at[qi, 0],
            device_id=(partner_b,),
            device_id_type=pl.DeviceIdType.MESH,
        )
        rb2 = pltpu.make_async_remote_copy(
            src_ref=sbuf_ref.at[qi, 1],
            dst_ref=rbuf2_ref.at[qi, 1],
            send_sem=s2_send.at[qi, 1],
            recv_sem=s2_recv.at[qi, 1],
            device_id=(partner_a,),
            device_id_type=pl.DeviceIdType.MESH,
        )
        ra2.start()
        rb2.start()
        stage2[qi] = (ra2, rb2)

    for qi in range(NB):
        r0 = qi * BQ
        for h in range(H_PER):
            q = q_ref[pl.ds(r0, BQ), pl.ds(h * DH, DH)]
            l = jnp.zeros((BQ, 1), jnp.float32)
            acc = jnp.zeros((BQ, DH), jnp.float32)
            for j in range(qi):
                kc = k_ref[h, pl.ds(j * BQ, BQ), :]
                sc = lax.dot_general(
                    q, kc, (((1,), (1,)), ((), ())),
                    preferred_element_type=jnp.float32,
                )
                pc = jnp.exp(sc)
                l = l + jnp.sum(pc, axis=1, keepdims=True)
                acc = acc + jnp.dot(
                    pc.astype(jnp.bfloat16),
                    v_ref[h, pl.ds(j * BQ, BQ), :],
                    preferred_element_type=jnp.float32,
                )
            kd = k_ref[h, pl.ds(r0, BQ), :]
            s_d = lax.dot_general(
                q, kd, (((1,), (1,)), ((), ())),
                preferred_element_type=jnp.float32,
            )
            rowb = lax.broadcasted_iota(jnp.int32, (BQ, BQ), 0) // BLK
            colb = lax.broadcasted_iota(jnp.int32, (BQ, BQ), 1) // BLK
            p_d = jnp.where(colb <= rowb, jnp.exp(s_d), 0.0)
            l = l + jnp.sum(p_d, axis=1, keepdims=True)
            acc = acc + jnp.dot(
                p_d.astype(jnp.bfloat16),
                v_ref[h, pl.ds(r0, BQ), :],
                preferred_element_type=jnp.float32,
            )
            ctx_ref[:, pl.ds(h * DH, DH)] = (acc / l).astype(jnp.bfloat16)

        p_ref[pl.ds(r0, BQ), :] = jnp.dot(
            ctx_ref[...], wo, preferred_element_type=jnp.float32
        ).astype(jnp.bfloat16)
        if qi == 0:
            pl.semaphore_wait(barrier, 2)
        start_stage1(qi)
        if qi > 0:
            finish1_start2(qi - 1)

    finish1_start2(NB - 1)

    for qi in range(NB):
        ra2, rb2 = stage2[qi]
        ra2.wait()
        rb2.wait()
        r0 = qi * BQ
        out_ref[pl.ds(r0, QH), :] = (sbuf_ref[qi, 0] + rbuf2_ref[qi, 0])
        out_ref[pl.ds(r0 + QH, QH), :] = (sbuf_ref[qi, 1] + rbuf2_ref[qi, 1])


def kernel(x, Wq, K_ext, V_ext, Wo):
    my = lax.axis_index("i")
    x2 = x.reshape(SQ, D_MODEL)
    my_idx = my.astype(jnp.int32).reshape((1,))

    Q, K, V = pl.pallas_call(
        _prep_body,
        grid_spec=pltpu.PrefetchScalarGridSpec(
            num_scalar_prefetch=1,
            grid=(SQ // BQ,),
            in_specs=[
                pl.BlockSpec((BQ, D_MODEL), lambda i, s: (i, 0)),
                pl.BlockSpec((D_MODEL, D_MODEL), lambda i, s: (0, 0)),
                pl.BlockSpec((BQ, H_PER, DH), lambda i, s: (i, s[0], 0)),
                pl.BlockSpec((BQ, H_PER, DH), lambda i, s: (i, s[0], 0)),
            ],
            out_specs=[
                pl.BlockSpec((BQ, D_MODEL), lambda i, s: (i, 0)),
                pl.BlockSpec((H_PER, BQ, DH), lambda i, s: (0, i, 0)),
                pl.BlockSpec((H_PER, BQ, DH), lambda i, s: (0, i, 0)),
            ],
        ),
        out_shape=[
            jax.ShapeDtypeStruct((SQ, D_MODEL), jnp.bfloat16),
            jax.ShapeDtypeStruct((H_PER, SKV, DH), jnp.bfloat16),
            jax.ShapeDtypeStruct((H_PER, SKV, DH), jnp.bfloat16),
        ],
    )(my_idx, x2, Wq, K_ext.reshape(SKV, 32, DH), V_ext.reshape(SKV, 32, DH))

    out = pl.pallas_call(
        _mega_body,
        out_shape=jax.ShapeDtypeStruct((SQ, D_MODEL), jnp.bfloat16),
        in_specs=[
            pl.BlockSpec(memory_space=pltpu.VMEM),
            pl.BlockSpec(memory_space=pltpu.VMEM),
            pl.BlockSpec(memory_space=pltpu.VMEM),
            pl.BlockSpec(memory_space=pltpu.VMEM),
        ],
        out_specs=pl.BlockSpec(memory_space=pltpu.VMEM),
        scratch_shapes=[
            pltpu.VMEM((BQ, H_PER * DH), jnp.bfloat16),
            pltpu.VMEM((SQ, D_MODEL), jnp.bfloat16),
            pltpu.VMEM((NB, 2, QH, D_MODEL), jnp.bfloat16),
            pltpu.VMEM((NB, 2, QH, D_MODEL), jnp.bfloat16),
            pltpu.VMEM((NB, 2, QH, D_MODEL), jnp.bfloat16),
            pltpu.SemaphoreType.DMA((NB, 2)),
            pltpu.SemaphoreType.DMA((NB, 2)),
            pltpu.SemaphoreType.DMA((NB, 2)),
            pltpu.SemaphoreType.DMA((NB, 2)),
        ],
        compiler_params=pltpu.CompilerParams(collective_id=0),
    )(Q, K, V, Wo)

    return out.reshape(1, SQ, D_MODEL)


# device time: 79093 ns/iter; 1.4638x vs baseline; 1.0014x over previous
import jax
import jax.numpy as jnp
from jax import lax
from jax.experimental import pallas as pl
from jax.experimental.pallas import tpu as pltpu

N_DEV = 4
SQ = 2048
SKV = 2048
D_MODEL = 1024
H_PER = 8
DH = 128
BQ = 512
NB = SQ // BQ
QH = BQ // 2
SCALE = 0.08838834764831843
BLK = 64


def _prep_body(idx_ref, x_ref, wq_ref, kslab_ref, vslab_ref,
               q_ref, k8_ref, v8_ref):
    x = x_ref[...].astype(jnp.bfloat16)
    wq = wq_ref[...].astype(jnp.bfloat16)
    q_ref[...] = (
        jnp.dot(x, wq, preferred_element_type=jnp.float32) * SCALE
    ).astype(jnp.bfloat16)
    k8_ref[...] = jnp.transpose(kslab_ref[...].astype(jnp.bfloat16), (1, 0, 2))
    v8_ref[...] = jnp.transpose(vslab_ref[...].astype(jnp.bfloat16), (1, 0, 2))


def _mega_body(q_ref, k_ref, v_ref, wo_ref, out_ref,
               ctx_ref, p_ref, sbuf_ref, rbuf1_ref, rbuf2_ref,
               s1_send, s1_recv, s2_send, s2_recv):
    my = lax.axis_index("i")
    partner_a = my ^ 1
    partner_b = 3 - my

    barrier = pltpu.get_barrier_semaphore()
    for nbr in (partner_a, partner_b):
        pl.semaphore_signal(
            barrier, inc=1, device_id=(nbr,),
            device_id_type=pl.DeviceIdType.MESH,
        )

    wo = wo_ref[...].astype(jnp.bfloat16)

    stage1 = {}
    stage2 = {}

    def start_stage1(qi):
        r0 = qi * BQ
        ra = pltpu.make_async_remote_copy(
            src_ref=p_ref.at[pl.ds(r0, QH), :],
            dst_ref=rbuf1_ref.at[qi, 0],
            send_sem=s1_send.at[qi, 0],
            recv_sem=s1_recv.at[qi, 0],
            device_id=(partner_a,),
            device_id_type=pl.DeviceIdType.MESH,
        )
        rb = pltpu.make_async_remote_copy(
            src_ref=p_ref.at[pl.ds(r0 + QH, QH), :],
            dst_ref=rbuf1_ref.at[qi, 1],
            send_sem=s1_send.at[qi, 1],
            recv_sem=s1_recv.at[qi, 1],
            device_id=(partner_b,),
            device_id_type=pl.DeviceIdType.MESH,
        )
        ra.start()
        rb.start()
        stage1[qi] = (ra, rb)

    def finish1_start2(qi):
        ra, rb = stage1[qi]
        ra.wait()
        rb.wait()
        r0 = qi * BQ
        sbuf_ref[qi, 0] = p_ref[pl.ds(r0, QH), :] + rbuf1_ref[qi, 0]
        sbuf_ref[qi, 1] = p_ref[pl.ds(r0 + QH, QH), :] + rbuf1_ref[qi, 1]
        ra2 = pltpu.make_async_remote_copy(
            src_ref=sbuf_ref.at[qi, 0],
            dst_ref=rbuf2_ref.at[qi, 0],
            send_sem=s2_send.at[qi, 0],
            recv_sem=s2_recv.at[qi, 0],
            device_id=(partner_b,),
            device_id_type=pl.DeviceIdType.MESH,
        )
        rb2 = pltpu.make_async_remote_copy(
            src_ref=sbuf_ref.at[qi, 1],
            dst_ref=rbuf2_ref.at[qi, 1],
            send_sem=s2_send.at[qi, 1],
            recv_sem=s2_recv.at[qi, 1],
            device_id=(partner_a,),
            device_id_type=pl.DeviceIdType.MESH,
        )
        ra2.start()
        rb2.start()
        stage2[qi] = (ra2, rb2)

    for qi in range(NB):
        r0 = qi * BQ
        for h in range(H_PER):
            q = q_ref[pl.ds(r0, BQ), pl.ds(h * DH, DH)]
            l = jnp.zeros((BQ, 1), jnp.float32)
            acc = jnp.zeros((BQ, DH), jnp.float32)
            for j in range(qi):
                kc = k_ref[h, pl.ds(j * BQ, BQ), :]
                sc = lax.dot_general(
                    q, kc, (((1,), (1,)), ((), ())),
                    preferred_element_type=jnp.float32,
                )
                pc = jnp.exp(sc)
                l = l + jnp.sum(pc, axis=1, keepdims=True)
                acc = acc + jnp.dot(
                    pc.astype(jnp.bfloat16),
                    v_ref[h, pl.ds(j * BQ, BQ), :],
                    preferred_element_type=jnp.float32,
                )
            HB = BQ // 2
            kd_t = k_ref[h, pl.ds(r0, HB), :]
            s_t = lax.dot_general(
                q[:HB], kd_t, (((1,), (1,)), ((), ())),
                preferred_element_type=jnp.float32,
            )
            rowb = lax.broadcasted_iota(jnp.int32, (HB, HB), 0) // BLK
            colb = lax.broadcasted_iota(jnp.int32, (HB, HB), 1) // BLK
            p_t = jnp.where(colb <= rowb, jnp.exp(s_t), 0.0)
            l_t = l[:HB] + jnp.sum(p_t, axis=1, keepdims=True)
            acc_t = acc[:HB] + jnp.dot(
                p_t.astype(jnp.bfloat16),
                v_ref[h, pl.ds(r0, HB), :],
                preferred_element_type=jnp.float32,
            )
            kd_b = k_ref[h, pl.ds(r0, BQ), :]
            s_b = lax.dot_general(
                q[HB:], kd_b, (((1,), (1,)), ((), ())),
                preferred_element_type=jnp.float32,
            )
            rowb2 = (lax.broadcasted_iota(jnp.int32, (HB, BQ), 0) + HB) // BLK
            colb2 = lax.broadcasted_iota(jnp.int32, (HB, BQ), 1) // BLK
            p_b = jnp.where(colb2 <= rowb2, jnp.exp(s_b), 0.0)
            l_b = l[HB:] + jnp.sum(p_b, axis=1, keepdims=True)
            acc_b = acc[HB:] + jnp.dot(
                p_b.astype(jnp.bfloat16),
                v_ref[h, pl.ds(r0, BQ), :],
                preferred_element_type=jnp.float32,
            )
            ctx_ref[pl.ds(0, HB), pl.ds(h * DH, DH)] = (
                acc_t / l_t
            ).astype(jnp.bfloat16)
            ctx_ref[pl.ds(HB, HB), pl.ds(h * DH, DH)] = (
                acc_b / l_b
            ).astype(jnp.bfloat16)

        p_ref[pl.ds(r0, BQ), :] = jnp.dot(
            ctx_ref[...], wo, preferred_element_type=jnp.float32
        ).astype(jnp.bfloat16)
        if qi == 0:
            pl.semaphore_wait(barrier, 2)
        start_stage1(qi)
        if qi > 0:
            finish1_start2(qi - 1)

    finish1_start2(NB - 1)

    for qi in range(NB):
        ra2, rb2 = stage2[qi]
        ra2.wait()
        rb2.wait()
        r0 = qi * BQ
        out_ref[pl.ds(r0, QH), :] = (sbuf_ref[qi, 0] + rbuf2_ref[qi, 0])
        out_ref[pl.ds(r0 + QH, QH), :] = (sbuf_ref[qi, 1] + rbuf2_ref[qi, 1])


def kernel(x, Wq, K_ext, V_ext, Wo):
    my = lax.axis_index("i")
    x2 = x.reshape(SQ, D_MODEL)
    my_idx = my.astype(jnp.int32).reshape((1,))

    Q, K, V = pl.pallas_call(
        _prep_body,
        grid_spec=pltpu.PrefetchScalarGridSpec(
            num_scalar_prefetch=1,
            grid=(SQ // BQ,),
            in_specs=[
                pl.BlockSpec((BQ, D_MODEL), lambda i, s: (i, 0)),
                pl.BlockSpec((D_MODEL, D_MODEL), lambda i, s: (0, 0)),
                pl.BlockSpec((BQ, H_PER, DH), lambda i, s: (i, s[0], 0)),
                pl.BlockSpec((BQ, H_PER, DH), lambda i, s: (i, s[0], 0)),
            ],
            out_specs=[
                pl.BlockSpec((BQ, D_MODEL), lambda i, s: (i, 0)),
                pl.BlockSpec((H_PER, BQ, DH), lambda i, s: (0, i, 0)),
                pl.BlockSpec((H_PER, BQ, DH), lambda i, s: (0, i, 0)),
            ],
        ),
        out_shape=[
            jax.ShapeDtypeStruct((SQ, D_MODEL), jnp.bfloat16),
            jax.ShapeDtypeStruct((H_PER, SKV, DH), jnp.bfloat16),
            jax.ShapeDtypeStruct((H_PER, SKV, DH), jnp.bfloat16),
        ],
    )(my_idx, x2, Wq, K_ext.reshape(SKV, 32, DH), V_ext.reshape(SKV, 32, DH))

    out = pl.pallas_call(
        _mega_body,
        out_shape=jax.ShapeDtypeStruct((SQ, D_MODEL), jnp.bfloat16),
        in_specs=[
            pl.BlockSpec(memory_space=pltpu.VMEM),
            pl.BlockSpec(memory_space=pltpu.VMEM),
            pl.BlockSpec(memory_space=pltpu.VMEM),
            pl.BlockSpec(memory_space=pltpu.VMEM),
        ],
        out_specs=pl.BlockSpec(memory_space=pltpu.VMEM),
        scratch_shapes=[
            pltpu.VMEM((BQ, H_PER * DH), jnp.bfloat16),
            pltpu.VMEM((SQ, D_MODEL), jnp.bfloat16),
            pltpu.VMEM((NB, 2, QH, D_MODEL), jnp.bfloat16),
            pltpu.VMEM((NB, 2, QH, D_MODEL), jnp.bfloat16),
            pltpu.VMEM((NB, 2, QH, D_MODEL), jnp.bfloat16),
            pltpu.SemaphoreType.DMA((NB, 2)),
            pltpu.SemaphoreType.DMA((NB, 2)),
            pltpu.SemaphoreType.DMA((NB, 2)),
            pltpu.SemaphoreType.DMA((NB, 2)),
        ],
        compiler_params=pltpu.CompilerParams(collective_id=0),
    )(Q, K, V, Wo)

    return out.reshape(1, SQ, D_MODEL)
